# baseline (speedup 1.0000x reference)
import math

import jax
import jax.numpy as jnp
from jax.experimental import pallas as pl
from jax.experimental.pallas import tpu as pltpu

_N, _A, _H, _W = 2, 3, 160, 160
_NUM = _A * _H * _W
_ROWS, _LANES = 600, 128
_PRE = 6000
_POST = 1000
_THRESH = 0.7
_IM_W, _IM_H = 800.0, 800.0
_MIN_SIZE = 0.0
_BBOX_CLIP = float(math.log(1000.0 / 16.0))


def _rpn_kernel(logit_ref, anc_ref, reg_ref,
                boxes_ref, scores_ref, mask_ref,
                x1_ref, y1_ref, x2_ref, y2_ref, area_ref, s_ref,
                bmax_ref, sx1_ref, sy1_ref, sx2_ref, sy2_ref, sarea_ref):
    logit = logit_ref[0]
    score = jax.nn.sigmoid(logit)

    row_iota = jax.lax.broadcasted_iota(jnp.int32, (_ROWS, _LANES), 0)
    col_iota = jax.lax.broadcasted_iota(jnp.int32, (_ROWS, _LANES), 1)
    iota = row_iota * _LANES + col_iota

    def _bis_body(_, carry):
        lo, hi = carry
        mid = 0.5 * (lo + hi)
        cnt = jnp.sum((score >= mid).astype(jnp.int32))
        take = cnt >= _PRE
        return jnp.where(take, mid, lo), jnp.where(take, hi, mid)

    lo, hi = jax.lax.fori_loop(
        0, 60, _bis_body, (jnp.float32(0.0), jnp.float32(1.0)))

    n_hi = jnp.sum((score >= hi).astype(jnp.int32))
    k = _PRE - n_hi
    ties = (score >= lo) & (score < hi)

    def _tie_body(_, carry):
        lo_t, hi_t = carry
        mid_t = (lo_t + hi_t) // 2
        cnt = jnp.sum((ties & (iota <= mid_t)).astype(jnp.int32))
        take = cnt >= k
        return jnp.where(take, lo_t, mid_t + 1), jnp.where(take, mid_t, hi_t)

    _, tie_T = jax.lax.fori_loop(
        0, 18, _tie_body, (jnp.int32(0), jnp.int32(_NUM - 1)))

    participate = (score >= hi) | (ties & (iota <= tie_T))
    s0 = jnp.where(participate, score, -1.0)

    ax1 = anc_ref[0, 0]
    ay1 = anc_ref[0, 1]
    ax2 = anc_ref[0, 2]
    ay2 = anc_ref[0, 3]
    dx = reg_ref[0, 0]
    dy = reg_ref[0, 1]
    dw = jnp.minimum(reg_ref[0, 2], _BBOX_CLIP)
    dh = jnp.minimum(reg_ref[0, 3], _BBOX_CLIP)

    widths = ax2 - ax1 + 1.0
    heights = ay2 - ay1 + 1.0
    ctr_x = ax1 + 0.5 * widths
    ctr_y = ay1 + 0.5 * heights
    pred_ctr_x = dx * widths + ctr_x
    pred_ctr_y = dy * heights + ctr_y
    pred_w = jnp.exp(dw) * widths
    pred_h = jnp.exp(dh) * heights

    x1 = jnp.clip(pred_ctr_x - 0.5 * pred_w, 0.0, _IM_W - 1.0)
    y1 = jnp.clip(pred_ctr_y - 0.5 * pred_h, 0.0, _IM_H - 1.0)
    x2 = jnp.clip(pred_ctr_x + 0.5 * pred_w - 1.0, 0.0, _IM_W - 1.0)
    y2 = jnp.clip(pred_ctr_y + 0.5 * pred_h - 1.0, 0.0, _IM_H - 1.0)

    ws = x2 - x1 + 1.0
    hs = y2 - y1 + 1.0
    keep = (ws >= _MIN_SIZE) & (hs >= _MIN_SIZE)
    s0 = jnp.where(keep, s0, -1.0)

    x1_ref[...] = x1
    y1_ref[...] = y1
    x2_ref[...] = x2
    y2_ref[...] = y2
    area_ref[...] = ws * hs
    s_ref[...] = s0

    lane_iota = jax.lax.broadcasted_iota(jnp.int32, (1, _LANES), 1)
    sub8 = jax.lax.broadcasted_iota(jnp.int32, (8, _LANES), 0)
    lane8 = jax.lax.broadcasted_iota(jnp.int32, (8, _LANES), 1)
    flat8 = sub8 * _LANES + lane8

    boxes_ref[...] = jnp.zeros((1, _POST, 4), jnp.float32)
    scores_ref[...] = jnp.zeros((1, _POST, 1), jnp.float32)
    mask_ref[...] = jnp.zeros((1, _POST, 1), jnp.float32)

    bmax_ref[...] = jnp.full((8, _LANES), -2.0, jnp.float32)

    def _bm_init(j, _):
        blk = s_ref[pl.ds(j * 8, 8), :]
        bmax_ref[...] = jnp.where(flat8 == j, jnp.max(blk), bmax_ref[...])
        return 0

    jax.lax.fori_loop(0, _ROWS // 8, _bm_init, 0)

    sx1_ref[...] = jnp.full((8, _LANES), 1e9, jnp.float32)
    sy1_ref[...] = jnp.full((8, _LANES), 1e9, jnp.float32)
    sx2_ref[...] = jnp.full((8, _LANES), -1e9, jnp.float32)
    sy2_ref[...] = jnp.full((8, _LANES), -1e9, jnp.float32)
    sarea_ref[...] = jnp.ones((8, _LANES), jnp.float32)

    def _cond(carry):
        _, cont = carry
        return cont != 0

    def _body(carry):
        i, _ = carry
        bm = bmax_ref[...]
        m = jnp.max(bm)
        b = jnp.min(jnp.where(bm == m, flat8, 1024))
        blk = s_ref[pl.ds(b * 8, 8), :]
        fidx = jnp.min(jnp.where(blk == m, flat8, 1024))
        r = b * 8 + fidx // _LANES
        c = fidx - (fidx // _LANES) * _LANES
        valid = m > 0.0

        def _extract(ref):
            row = ref[pl.ds(r, 1), :]
            return jnp.sum(jnp.where(lane_iota == c, row, 0.0))

        bx1 = _extract(x1_ref)
        by1 = _extract(y1_ref)
        bx2 = _extract(x2_ref)
        by2 = _extract(y2_ref)
        barea = _extract(area_ref)

        xx1 = jnp.maximum(bx1, sx1_ref[...])
        yy1 = jnp.maximum(by1, sy1_ref[...])
        xx2 = jnp.minimum(bx2, sx2_ref[...])
        yy2 = jnp.minimum(by2, sy2_ref[...])
        w = jnp.maximum(xx2 - xx1 + 1.0, 0.0)
        h = jnp.maximum(yy2 - yy1 + 1.0, 0.0)
        inter = w * h
        iou = inter / (barea + sarea_ref[...] - inter)
        suppressed = jnp.max(jnp.where(iou > _THRESH, 1.0, 0.0)) > 0.0
        keep = valid & jnp.logical_not(suppressed)

        row = s_ref[pl.ds(r, 1), :]
        s_ref[pl.ds(r, 1), :] = jnp.where(lane_iota == c, -1.0, row)
        blk2 = s_ref[pl.ds(b * 8, 8), :]
        bmax_ref[...] = jnp.where(flat8 == b, jnp.max(blk2), bm)

        app = keep & (flat8 == i)
        sx1_ref[...] = jnp.where(app, bx1, sx1_ref[...])
        sy1_ref[...] = jnp.where(app, by1, sy1_ref[...])
        sx2_ref[...] = jnp.where(app, bx2, sx2_ref[...])
        sy2_ref[...] = jnp.where(app, by2, sy2_ref[...])
        sarea_ref[...] = jnp.where(app, barea, sarea_ref[...])

        ci4 = jax.lax.broadcasted_iota(jnp.int32, (1, 4), 1)
        box_row = jnp.where(ci4 == 0, bx1,
                  jnp.where(ci4 == 1, by1,
                  jnp.where(ci4 == 2, bx2, by2)))
        old_box = boxes_ref[0, pl.ds(i, 1), :]
        boxes_ref[0, pl.ds(i, 1), :] = jnp.where(keep, box_row, old_box)
        old_s = scores_ref[0, pl.ds(i, 1), :]
        scores_ref[0, pl.ds(i, 1), :] = jnp.where(
            keep, jnp.full((1, 1), m, jnp.float32), old_s)
        old_m = mask_ref[0, pl.ds(i, 1), :]
        mask_ref[0, pl.ds(i, 1), :] = jnp.where(
            keep, jnp.ones((1, 1), jnp.float32), old_m)

        i_next = i + keep.astype(jnp.int32)
        cont = (valid & (i_next < _POST)).astype(jnp.int32)
        return i_next, cont

    jax.lax.while_loop(_cond, _body, (jnp.int32(0), jnp.int32(1)))


def kernel(anchors, objectness, box_regression):
    obj = jnp.transpose(objectness, (0, 2, 3, 1)).reshape(_N, _ROWS, _LANES)
    reg = box_regression.reshape(_N, _A, 4, _H, _W)
    reg = jnp.transpose(reg, (0, 3, 4, 1, 2)).reshape(_N, _NUM, 4)
    reg = jnp.transpose(reg, (0, 2, 1)).reshape(_N, 4, _ROWS, _LANES)
    anc = jnp.transpose(anchors.reshape(_N, _NUM, 4), (0, 2, 1))
    anc = anc.reshape(_N, 4, _ROWS, _LANES)

    boxes, scores, mask = pl.pallas_call(
        _rpn_kernel,
        grid=(_N,),
        in_specs=[
            pl.BlockSpec((1, _ROWS, _LANES), lambda n: (n, 0, 0)),
            pl.BlockSpec((1, 4, _ROWS, _LANES), lambda n: (n, 0, 0, 0)),
            pl.BlockSpec((1, 4, _ROWS, _LANES), lambda n: (n, 0, 0, 0)),
        ],
        out_specs=[
            pl.BlockSpec((1, _POST, 4), lambda n: (n, 0, 0)),
            pl.BlockSpec((1, _POST, 1), lambda n: (n, 0, 0)),
            pl.BlockSpec((1, _POST, 1), lambda n: (n, 0, 0)),
        ],
        out_shape=[
            jax.ShapeDtypeStruct((_N, _POST, 4), jnp.float32),
            jax.ShapeDtypeStruct((_N, _POST, 1), jnp.float32),
            jax.ShapeDtypeStruct((_N, _POST, 1), jnp.float32),
        ],
        scratch_shapes=(
            [pltpu.VMEM((_ROWS, _LANES), jnp.float32) for _ in range(6)]
            + [pltpu.VMEM((8, _LANES), jnp.float32) for _ in range(6)]),
        compiler_params=pltpu.CompilerParams(
            dimension_semantics=("arbitrary",)),
    )(obj, anc, reg)

    return boxes, scores.reshape(_N, _POST), mask.reshape(_N, _POST)


# speedup vs baseline: 1.0720x; 1.0720x over previous
import math

import jax
import jax.numpy as jnp
from jax.experimental import pallas as pl
from jax.experimental.pallas import tpu as pltpu

_N, _A, _H, _W = 2, 3, 160, 160
_NUM = _A * _H * _W
_ROWS, _LANES = 600, 128
_NBLK = _ROWS // 8
_PRE = 6000
_POST = 1000
_THRESH = 0.7
_IM_W, _IM_H = 800.0, 800.0
_MIN_SIZE = 0.0
_BBOX_CLIP = float(math.log(1000.0 / 16.0))


def _rpn_kernel(logit_ref, anc_ref, reg_ref,
                boxes_ref, scores_ref, mask_ref,
                x1_ref, y1_ref, x2_ref, y2_ref, area_ref, s_ref,
                bmax_ref, sx1_ref, sy1_ref, sx2_ref, sy2_ref, sarea_ref):
    sub8 = jax.lax.broadcasted_iota(jnp.int32, (8, _LANES), 0)
    lane8 = jax.lax.broadcasted_iota(jnp.int32, (8, _LANES), 1)
    flat8 = sub8 * _LANES + lane8
    ci4 = jax.lax.broadcasted_iota(jnp.int32, (1, 4), 1)

    row_iota = jax.lax.broadcasted_iota(jnp.int32, (_ROWS, _LANES), 0)
    col_iota = jax.lax.broadcasted_iota(jnp.int32, (_ROWS, _LANES), 1)
    iota = row_iota * _LANES + col_iota

    for n in range(_N):
        logit = logit_ref[n]
        score = jax.nn.sigmoid(logit)

        def _bis_body(_, carry, score=score):
            lo, hi = carry
            mid = 0.5 * (lo + hi)
            cnt = jnp.sum((score >= mid).astype(jnp.int32))
            take = cnt >= _PRE
            return jnp.where(take, mid, lo), jnp.where(take, hi, mid)

        lo, hi = jax.lax.fori_loop(
            0, 60, _bis_body, (jnp.float32(0.0), jnp.float32(1.0)))

        n_hi = jnp.sum((score >= hi).astype(jnp.int32))
        k = _PRE - n_hi
        ties = (score >= lo) & (score < hi)

        def _tie_body(_, carry, ties=ties, k=k):
            lo_t, hi_t = carry
            mid_t = (lo_t + hi_t) // 2
            cnt = jnp.sum((ties & (iota <= mid_t)).astype(jnp.int32))
            take = cnt >= k
            return jnp.where(take, lo_t, mid_t + 1), jnp.where(take, mid_t, hi_t)

        _, tie_T = jax.lax.fori_loop(
            0, 18, _tie_body, (jnp.int32(0), jnp.int32(_NUM - 1)))

        participate = (score >= hi) | (ties & (iota <= tie_T))
        s0 = jnp.where(participate, score, -1.0)

        ax1 = anc_ref[n, 0]
        ay1 = anc_ref[n, 1]
        ax2 = anc_ref[n, 2]
        ay2 = anc_ref[n, 3]
        dx = reg_ref[n, 0]
        dy = reg_ref[n, 1]
        dw = jnp.minimum(reg_ref[n, 2], _BBOX_CLIP)
        dh = jnp.minimum(reg_ref[n, 3], _BBOX_CLIP)

        widths = ax2 - ax1 + 1.0
        heights = ay2 - ay1 + 1.0
        ctr_x = ax1 + 0.5 * widths
        ctr_y = ay1 + 0.5 * heights
        pred_ctr_x = dx * widths + ctr_x
        pred_ctr_y = dy * heights + ctr_y
        pred_w = jnp.exp(dw) * widths
        pred_h = jnp.exp(dh) * heights

        x1 = jnp.clip(pred_ctr_x - 0.5 * pred_w, 0.0, _IM_W - 1.0)
        y1 = jnp.clip(pred_ctr_y - 0.5 * pred_h, 0.0, _IM_H - 1.0)
        x2 = jnp.clip(pred_ctr_x + 0.5 * pred_w - 1.0, 0.0, _IM_W - 1.0)
        y2 = jnp.clip(pred_ctr_y + 0.5 * pred_h - 1.0, 0.0, _IM_H - 1.0)

        ws = x2 - x1 + 1.0
        hs = y2 - y1 + 1.0
        keep0 = (ws >= _MIN_SIZE) & (hs >= _MIN_SIZE)
        s0 = jnp.where(keep0, s0, -1.0)

        x1_ref[n] = x1
        y1_ref[n] = y1
        x2_ref[n] = x2
        y2_ref[n] = y2
        area_ref[n] = ws * hs
        s_ref[n] = s0

        bmax_ref[n] = jnp.full((8, _LANES), -2.0, jnp.float32)

        def _bm_init(j, _, n=n):
            blk = s_ref[n, pl.ds(j * 8, 8), :]
            bmax_ref[n] = jnp.where(flat8 == j, jnp.max(blk), bmax_ref[n])
            return 0

        jax.lax.fori_loop(0, _NBLK, _bm_init, 0)

        sx1_ref[n] = jnp.full((8, _LANES), 1e9, jnp.float32)
        sy1_ref[n] = jnp.full((8, _LANES), 1e9, jnp.float32)
        sx2_ref[n] = jnp.full((8, _LANES), -1e9, jnp.float32)
        sy2_ref[n] = jnp.full((8, _LANES), -1e9, jnp.float32)
        sarea_ref[n] = jnp.ones((8, _LANES), jnp.float32)

        boxes_ref[n] = jnp.zeros((_POST, 4), jnp.float32)
        scores_ref[n] = jnp.zeros((_POST, 1), jnp.float32)
        mask_ref[n] = jnp.zeros((_POST, 1), jnp.float32)

    def _cond(carry):
        _, cont0, _, cont1 = carry
        return (cont0 != 0) | (cont1 != 0)

    def _pop(n, i, cont):
        bm = bmax_ref[n]
        m = jnp.max(bm)
        b = jnp.min(jnp.where(bm == m, flat8, 1024))
        blk = s_ref[n, pl.ds(b * 8, 8), :]
        fidx = jnp.min(jnp.where(blk == m, flat8, 1024))
        valid = (m > 0.0) & (cont != 0)

        def _extract(ref):
            v = ref[n, pl.ds(b * 8, 8), :]
            return jnp.sum(jnp.where(flat8 == fidx, v, 0.0))

        bx1 = _extract(x1_ref)
        by1 = _extract(y1_ref)
        bx2 = _extract(x2_ref)
        by2 = _extract(y2_ref)
        barea = _extract(area_ref)

        xx1 = jnp.maximum(bx1, sx1_ref[n])
        yy1 = jnp.maximum(by1, sy1_ref[n])
        xx2 = jnp.minimum(bx2, sx2_ref[n])
        yy2 = jnp.minimum(by2, sy2_ref[n])
        w = jnp.maximum(xx2 - xx1 + 1.0, 0.0)
        h = jnp.maximum(yy2 - yy1 + 1.0, 0.0)
        inter = w * h
        iou = inter / (barea + sarea_ref[n] - inter)
        suppressed = jnp.max(jnp.where(iou > _THRESH, 1.0, 0.0)) > 0.0
        keep = valid & jnp.logical_not(suppressed)

        new_blk = jnp.where(flat8 == fidx, -1.0, blk)
        s_ref[n, pl.ds(b * 8, 8), :] = jnp.where(valid, new_blk, blk)
        bmax_ref[n] = jnp.where((flat8 == b) & valid, jnp.max(new_blk), bm)

        app = keep & (flat8 == i)
        sx1_ref[n] = jnp.where(app, bx1, sx1_ref[n])
        sy1_ref[n] = jnp.where(app, by1, sy1_ref[n])
        sx2_ref[n] = jnp.where(app, bx2, sx2_ref[n])
        sy2_ref[n] = jnp.where(app, by2, sy2_ref[n])
        sarea_ref[n] = jnp.where(app, barea, sarea_ref[n])

        box_row = jnp.where(ci4 == 0, bx1,
                  jnp.where(ci4 == 1, by1,
                  jnp.where(ci4 == 2, bx2, by2)))
        i_s = jnp.minimum(i, _POST - 1)
        old_box = boxes_ref[n, pl.ds(i_s, 1), :]
        boxes_ref[n, pl.ds(i_s, 1), :] = jnp.where(keep, box_row, old_box)
        old_s = scores_ref[n, pl.ds(i_s, 1), :]
        scores_ref[n, pl.ds(i_s, 1), :] = jnp.where(
            keep, jnp.full((1, 1), m, jnp.float32), old_s)
        old_m = mask_ref[n, pl.ds(i_s, 1), :]
        mask_ref[n, pl.ds(i_s, 1), :] = jnp.where(
            keep, jnp.ones((1, 1), jnp.float32), old_m)

        i_next = i + keep.astype(jnp.int32)
        cont_next = jnp.where(
            cont != 0, ((m > 0.0) & (i_next < _POST)).astype(jnp.int32), 0)
        return i_next, cont_next

    def _body(carry):
        i0, cont0, i1, cont1 = carry
        i0n, c0n = _pop(0, i0, cont0)
        i1n, c1n = _pop(1, i1, cont1)
        return i0n, c0n, i1n, c1n

    jax.lax.while_loop(
        _cond, _body,
        (jnp.int32(0), jnp.int32(1), jnp.int32(0), jnp.int32(1)))


def kernel(anchors, objectness, box_regression):
    obj = jnp.transpose(objectness, (0, 2, 3, 1)).reshape(_N, _ROWS, _LANES)
    reg = box_regression.reshape(_N, _A, 4, _H, _W)
    reg = jnp.transpose(reg, (0, 3, 4, 1, 2)).reshape(_N, _NUM, 4)
    reg = jnp.transpose(reg, (0, 2, 1)).reshape(_N, 4, _ROWS, _LANES)
    anc = jnp.transpose(anchors.reshape(_N, _NUM, 4), (0, 2, 1))
    anc = anc.reshape(_N, 4, _ROWS, _LANES)

    boxes, scores, mask = pl.pallas_call(
        _rpn_kernel,
        out_shape=[
            jax.ShapeDtypeStruct((_N, _POST, 4), jnp.float32),
            jax.ShapeDtypeStruct((_N, _POST, 1), jnp.float32),
            jax.ShapeDtypeStruct((_N, _POST, 1), jnp.float32),
        ],
        scratch_shapes=(
            [pltpu.VMEM((_N, _ROWS, _LANES), jnp.float32) for _ in range(6)]
            + [pltpu.VMEM((_N, 8, _LANES), jnp.float32) for _ in range(6)]),
    )(obj, anc, reg)

    return boxes, scores.reshape(_N, _POST), mask.reshape(_N, _POST)


# speedup vs baseline: 1.0746x; 1.0024x over previous
import math

import jax
import jax.numpy as jnp
from jax.experimental import pallas as pl
from jax.experimental.pallas import tpu as pltpu

_N, _A, _H, _W = 2, 3, 160, 160
_NUM = _A * _H * _W
_ROWS, _LANES = 600, 128
_NBLK = _ROWS // 8
_PRE = 6000
_POST = 1000
_THRESH = 0.7
_IM_W, _IM_H = 800.0, 800.0
_MIN_SIZE = 0.0
_BBOX_CLIP = float(math.log(1000.0 / 16.0))


def _rpn_kernel(logit_ref, anc_ref, reg_ref,
                bpl_ref, spl_ref, mpl_ref,
                x1_ref, y1_ref, x2_ref, y2_ref, s_ref):
    sub8 = jax.lax.broadcasted_iota(jnp.int32, (8, _LANES), 0)
    lane8 = jax.lax.broadcasted_iota(jnp.int32, (8, _LANES), 1)
    flat8 = sub8 * _LANES + lane8

    row_iota = jax.lax.broadcasted_iota(jnp.int32, (_ROWS, _LANES), 0)
    col_iota = jax.lax.broadcasted_iota(jnp.int32, (_ROWS, _LANES), 1)
    iota = row_iota * _LANES + col_iota

    bm_init = []
    for n in range(_N):
        logit = logit_ref[n]
        score = jax.nn.sigmoid(logit)

        def _bis_body(_, carry, score=score):
            lo, hi = carry
            mid = 0.5 * (lo + hi)
            cnt = jnp.sum((score >= mid).astype(jnp.int32))
            take = cnt >= _PRE
            return jnp.where(take, mid, lo), jnp.where(take, hi, mid)

        lo, hi = jax.lax.fori_loop(
            0, 60, _bis_body, (jnp.float32(0.0), jnp.float32(1.0)))

        n_hi = jnp.sum((score >= hi).astype(jnp.int32))
        k = _PRE - n_hi
        ties = (score >= lo) & (score < hi)

        def _tie_body(_, carry, ties=ties, k=k):
            lo_t, hi_t = carry
            mid_t = (lo_t + hi_t) // 2
            cnt = jnp.sum((ties & (iota <= mid_t)).astype(jnp.int32))
            take = cnt >= k
            return jnp.where(take, lo_t, mid_t + 1), jnp.where(take, mid_t, hi_t)

        _, tie_T = jax.lax.fori_loop(
            0, 18, _tie_body, (jnp.int32(0), jnp.int32(_NUM - 1)))

        participate = (score >= hi) | (ties & (iota <= tie_T))
        s0 = jnp.where(participate, score, -1.0)

        ax1 = anc_ref[n, 0]
        ay1 = anc_ref[n, 1]
        ax2 = anc_ref[n, 2]
        ay2 = anc_ref[n, 3]
        dx = reg_ref[n, 0]
        dy = reg_ref[n, 1]
        dw = jnp.minimum(reg_ref[n, 2], _BBOX_CLIP)
        dh = jnp.minimum(reg_ref[n, 3], _BBOX_CLIP)

        widths = ax2 - ax1 + 1.0
        heights = ay2 - ay1 + 1.0
        ctr_x = ax1 + 0.5 * widths
        ctr_y = ay1 + 0.5 * heights
        pred_ctr_x = dx * widths + ctr_x
        pred_ctr_y = dy * heights + ctr_y
        pred_w = jnp.exp(dw) * widths
        pred_h = jnp.exp(dh) * heights

        x1 = jnp.clip(pred_ctr_x - 0.5 * pred_w, 0.0, _IM_W - 1.0)
        y1 = jnp.clip(pred_ctr_y - 0.5 * pred_h, 0.0, _IM_H - 1.0)
        x2 = jnp.clip(pred_ctr_x + 0.5 * pred_w - 1.0, 0.0, _IM_W - 1.0)
        y2 = jnp.clip(pred_ctr_y + 0.5 * pred_h - 1.0, 0.0, _IM_H - 1.0)

        ws = x2 - x1 + 1.0
        hs = y2 - y1 + 1.0
        keep0 = (ws >= _MIN_SIZE) & (hs >= _MIN_SIZE)
        s0 = jnp.where(keep0, s0, -1.0)

        x1_ref[n] = x1
        y1_ref[n] = y1
        x2_ref[n] = x2
        y2_ref[n] = y2
        s_ref[n] = s0

        def _bm_body(j, bm, n=n):
            blk = s_ref[n, pl.ds(j * 8, 8), :]
            return jnp.where(flat8 == j, jnp.max(blk), bm)

        bm_init.append(jax.lax.fori_loop(
            0, _NBLK, _bm_body, jnp.full((8, _LANES), -2.0, jnp.float32)))

    def _mk_state(bm):
        big = jnp.full((8, _LANES), 1e9, jnp.float32)
        return (jnp.int32(0), jnp.int32(1), bm,
                big, big, -big, -big,
                jnp.ones((8, _LANES), jnp.float32),
                jnp.zeros((8, _LANES), jnp.float32))

    def _cond(carry):
        st0, st1 = carry
        return (st0[1] != 0) | (st1[1] != 0)

    def _pop(n, st):
        i, cont, bm, sx1, sy1, sx2, sy2, sarea, ssc = st
        m = jnp.max(bm)
        b = jnp.min(jnp.where(bm == m, flat8, 1024))
        blk = s_ref[n, pl.ds(b * 8, 8), :]
        fidx = jnp.min(jnp.where(blk == m, flat8, 1024))
        valid = (m > 0.0) & (cont != 0)

        def _extract(ref):
            v = ref[n, pl.ds(b * 8, 8), :]
            return jnp.sum(jnp.where(flat8 == fidx, v, 0.0))

        bx1 = _extract(x1_ref)
        by1 = _extract(y1_ref)
        bx2 = _extract(x2_ref)
        by2 = _extract(y2_ref)
        barea = (bx2 - bx1 + 1.0) * (by2 - by1 + 1.0)

        xx1 = jnp.maximum(bx1, sx1)
        yy1 = jnp.maximum(by1, sy1)
        xx2 = jnp.minimum(bx2, sx2)
        yy2 = jnp.minimum(by2, sy2)
        w = jnp.maximum(xx2 - xx1 + 1.0, 0.0)
        h = jnp.maximum(yy2 - yy1 + 1.0, 0.0)
        inter = w * h
        iou = inter / (barea + sarea - inter)
        suppressed = jnp.max(jnp.where(iou > _THRESH, 1.0, 0.0)) > 0.0
        keep = valid & jnp.logical_not(suppressed)

        new_blk = jnp.where(flat8 == fidx, -1.0, blk)
        s_ref[n, pl.ds(b * 8, 8), :] = jnp.where(valid, new_blk, blk)
        bm = jnp.where((flat8 == b) & valid, jnp.max(new_blk), bm)

        app = keep & (flat8 == i)
        sx1 = jnp.where(app, bx1, sx1)
        sy1 = jnp.where(app, by1, sy1)
        sx2 = jnp.where(app, bx2, sx2)
        sy2 = jnp.where(app, by2, sy2)
        sarea = jnp.where(app, barea, sarea)
        ssc = jnp.where(app, m, ssc)

        i = i + keep.astype(jnp.int32)
        cont = jnp.where(cont != 0,
                         ((m > 0.0) & (i < _POST)).astype(jnp.int32), 0)
        return (i, cont, bm, sx1, sy1, sx2, sy2, sarea, ssc)

    def _body(carry):
        st0, st1 = carry
        return _pop(0, st0), _pop(1, st1)

    st0, st1 = jax.lax.while_loop(
        _cond, _body, (_mk_state(bm_init[0]), _mk_state(bm_init[1])))

    for n, st in ((0, st0), (1, st1)):
        i, _, _, sx1, sy1, sx2, sy2, _, ssc = st
        live = flat8 < i
        zero = jnp.zeros((8, _LANES), jnp.float32)
        bpl_ref[n, 0] = jnp.where(live, sx1, zero)
        bpl_ref[n, 1] = jnp.where(live, sy1, zero)
        bpl_ref[n, 2] = jnp.where(live, sx2, zero)
        bpl_ref[n, 3] = jnp.where(live, sy2, zero)
        spl_ref[n] = jnp.where(live, ssc, zero)
        mpl_ref[n] = jnp.where(live, jnp.ones((8, _LANES), jnp.float32), zero)


def kernel(anchors, objectness, box_regression):
    obj = jnp.transpose(objectness, (0, 2, 3, 1)).reshape(_N, _ROWS, _LANES)
    reg = box_regression.reshape(_N, _A, 4, _H, _W)
    reg = jnp.transpose(reg, (0, 3, 4, 1, 2)).reshape(_N, _NUM, 4)
    reg = jnp.transpose(reg, (0, 2, 1)).reshape(_N, 4, _ROWS, _LANES)
    anc = jnp.transpose(anchors.reshape(_N, _NUM, 4), (0, 2, 1))
    anc = anc.reshape(_N, 4, _ROWS, _LANES)

    bpl, spl, mpl = pl.pallas_call(
        _rpn_kernel,
        out_shape=[
            jax.ShapeDtypeStruct((_N, 4, 8, _LANES), jnp.float32),
            jax.ShapeDtypeStruct((_N, 8, _LANES), jnp.float32),
            jax.ShapeDtypeStruct((_N, 8, _LANES), jnp.float32),
        ],
        scratch_shapes=[pltpu.VMEM((_N, _ROWS, _LANES), jnp.float32)
                        for _ in range(5)],
    )(obj, anc, reg)

    boxes = jnp.transpose(bpl.reshape(_N, 4, 1024), (0, 2, 1))[:, :_POST, :]
    scores = spl.reshape(_N, 1024)[:, :_POST]
    mask = mpl.reshape(_N, 1024)[:, :_POST]
    return boxes, scores, mask
